# Initial kernel scaffold; baseline (speedup 1.0000x reference)
import jax
import jax.numpy as jnp
from jax.experimental import pallas as pl


def kernel(ck):
    raise NotImplementedError("write your pallas kernel here")


# speedup vs baseline: 15.2530x; 15.2530x over previous
import jax
import jax.numpy as jnp
from jax.experimental import pallas as pl
from jax.experimental.pallas import tpu as pltpu

_K = 64


def _topk_mask_body(ck_ref, out_ref):
    int_min = jnp.int32(-2147483648)
    x = ck_ref[...]
    n = x.shape[-1]
    i = jax.lax.bitcast_convert_type(x, jnp.int32)
    key = jnp.where(i < 0, i ^ jnp.int32(0x7FFFFFFF), i)

    cnt_pos = jnp.sum((key >= 0).astype(jnp.int32), axis=1, keepdims=True)
    t = jnp.where(cnt_pos >= _K, jnp.int32(0), int_min)

    def step(b, t):
        t_try = t | (jnp.int32(1) << (30 - b))
        cnt = jnp.sum((key >= t_try).astype(jnp.int32), axis=1, keepdims=True)
        return jnp.where(cnt >= _K, t_try, t)

    t = jax.lax.fori_loop(0, 31, step, t, unroll=True)

    gt = key > t
    cnt_gt = jnp.sum(gt.astype(jnp.int32), axis=1, keepdims=True)
    r = _K - cnt_gt

    idx = jax.lax.broadcasted_iota(jnp.int32, x.shape, 1)
    key2 = jnp.where(key == t, jnp.int32(n - 1) - idx, jnp.int32(-1))

    def step2(b, t2):
        t_try = t2 | (jnp.int32(1) << (12 - b))
        cnt = jnp.sum((key2 >= t_try).astype(jnp.int32), axis=1, keepdims=True)
        return jnp.where(cnt >= r, t_try, t2)

    t2 = jax.lax.fori_loop(0, 13, step2, jnp.zeros_like(t), unroll=True)

    out_ref[...] = (gt | (key2 >= t2)).astype(jnp.float32)


def kernel(ck):
    return pl.pallas_call(
        _topk_mask_body,
        out_shape=jax.ShapeDtypeStruct(ck.shape, jnp.float32),
    )(ck)
